# baseline (speedup 1.0000x reference)
import functools

import jax
import jax.numpy as jnp
from jax import lax
from jax.experimental import pallas as pl
from jax.experimental.pallas import tpu as pltpu
from jax.experimental.pallas import tpu_sc as plsc

N_EMBD = 768
NUM_EXPERTS = 64
TOP_K = 8
TOKENS = 32768
BLOCK = 4096

NC, NS, L = 2, 16, 16
NW = NC * NS
TOK_PER_W = TOKENS // NW
SLAB = 256
NSLAB = TOK_PER_W // SLAB
NGROUP = SLAB // L

_IMASK = NUM_EXPERTS - 1

_NET8 = ((0, 1), (2, 3), (4, 5), (6, 7), (0, 2), (1, 3), (4, 6), (5, 7),
         (1, 2), (5, 6), (0, 4), (3, 7), (1, 5), (2, 6), (1, 4), (3, 6),
         (2, 4), (3, 5), (3, 4))


def _keys_body(x_ref, w_ref, keys_ref):
    logits_t = jax.lax.dot_general(
        w_ref[...], x_ref[...],
        dimension_numbers=(((1,), (1,)), ((), ())),
        preferred_element_type=jnp.float32,
    )
    si = jax.lax.bitcast_convert_type(logits_t, jnp.int32)
    sortable = si ^ (jax.lax.shift_right_arithmetic(si, 31) & 0x7FFFFFFF)
    rev_iota = _IMASK - jax.lax.broadcasted_iota(
        jnp.int32, (NUM_EXPERTS, BLOCK), 0)
    keys_ref[...] = (sortable & ~_IMASK) | rev_iota


def _tc_keys(x, w_gate):
    return pl.pallas_call(
        _keys_body,
        grid=(TOKENS // BLOCK,),
        in_specs=[
            pl.BlockSpec((BLOCK, N_EMBD), lambda i: (i, 0)),
            pl.BlockSpec((NUM_EXPERTS, N_EMBD), lambda i: (0, 0)),
        ],
        out_specs=pl.BlockSpec((NUM_EXPERTS, BLOCK), lambda i: (0, i)),
        out_shape=jax.ShapeDtypeStruct((NUM_EXPERTS, TOKENS), jnp.int32),
    )(x, w_gate)


_SC_MESH = plsc.VectorSubcoreMesh(
    core_axis_name="c", subcore_axis_name="s", num_cores=NC, num_subcores=NS)


@functools.partial(
    pl.kernel,
    out_type=(
        jax.ShapeDtypeStruct((TOP_K, TOKENS), jnp.int32),
        jax.ShapeDtypeStruct((TOP_K, TOKENS), jnp.float32),
    ),
    mesh=_SC_MESH,
    scratch_types=[
        pltpu.VMEM((2, NUM_EXPERTS, SLAB), jnp.int32),
        pltpu.VMEM((TOP_K, TOK_PER_W), jnp.int32),
        pltpu.VMEM((TOP_K, TOK_PER_W), jnp.float32),
        pltpu.SemaphoreType.DMA,
        pltpu.SemaphoreType.DMA,
    ],
)
def _sc_topk(keys_hbm, idx_hbm, score_hbm, keys_v, idx_v, score_v,
             sem0, sem1):
    wid = lax.axis_index("s") * NC + lax.axis_index("c")
    base = wid * TOK_PER_W
    sems = (sem0, sem1)

    def start_slab(s):
        return pltpu.async_copy(
            keys_hbm.at[:, pl.ds(base + s * SLAB, SLAB)],
            keys_v.at[s % 2], sems[s % 2])

    copies = {0: start_slab(0)}
    for s in range(NSLAB):
        copies[s].wait()
        if s + 1 < NSLAB:
            copies[s + 1] = start_slab(s + 1)
        buf = s % 2

        def group(g, carry, buf=buf, s=s):
            for half in range(2):
                off = (2 * g + half) * L
                out_off = s * SLAB + off
                best = [keys_v[buf, e, pl.ds(off, L)] for e in range(TOP_K)]
                for (a, b) in _NET8:
                    hi = jnp.maximum(best[a], best[b])
                    best[b] = jnp.minimum(best[a], best[b])
                    best[a] = hi
                for e in range(TOP_K, NUM_EXPERTS):
                    v = keys_v[buf, e, pl.ds(off, L)]
                    for j in range(TOP_K):
                        hi = jnp.maximum(best[j], v)
                        v = jnp.minimum(best[j], v)
                        best[j] = hi
                vals = []
                for j in range(TOP_K):
                    k = best[j]
                    idx_v[j, pl.ds(out_off, L)] = _IMASK - (k & _IMASK)
                    vs = k & ~_IMASK
                    vsi = vs ^ (
                        lax.shift_right_arithmetic(vs, 31) & 0x7FFFFFFF)
                    vals.append(lax.bitcast_convert_type(vsi, jnp.float32))
                exps = [jnp.exp(v - vals[0]) for v in vals]
                tot = exps[0]
                for j in range(1, TOP_K):
                    tot = tot + exps[j]
                for j in range(TOP_K):
                    score_v[j, pl.ds(out_off, L)] = exps[j] / tot
            return carry

        lax.fori_loop(0, NGROUP // 2, group, 0)

    pltpu.sync_copy(idx_v, idx_hbm.at[:, pl.ds(base, TOK_PER_W)])
    pltpu.sync_copy(score_v, score_hbm.at[:, pl.ds(base, TOK_PER_W)])


@jax.jit
def kernel(x, w_gate):
    keys = _tc_keys(x, w_gate)
    idx_t, score_t = _sc_topk(keys)
    return idx_t.T, score_t.T


# speedup vs baseline: 1.0129x; 1.0129x over previous
import functools

import jax
import jax.numpy as jnp
from jax import lax
from jax.experimental import pallas as pl
from jax.experimental.pallas import tpu as pltpu
from jax.experimental.pallas import tpu_sc as plsc

N_EMBD = 768
NUM_EXPERTS = 64
TOP_K = 8
TOKENS = 32768
BLOCK = 4096

NC, NS, L = 2, 16, 16
NW = NC * NS
TOK_PER_W = TOKENS // NW
SLAB = 256
NSLAB = TOK_PER_W // SLAB
NGROUP = SLAB // L

_IMASK = NUM_EXPERTS - 1

_NET8 = ((0, 1), (2, 3), (4, 5), (6, 7), (0, 2), (1, 3), (4, 6), (5, 7),
         (1, 2), (5, 6), (0, 4), (3, 7), (1, 5), (2, 6), (1, 4), (3, 6),
         (2, 4), (3, 5), (3, 4))


def _keys_body(x_ref, w_ref, keys_ref):
    logits_t = jax.lax.dot_general(
        w_ref[...], x_ref[...],
        dimension_numbers=(((1,), (1,)), ((), ())),
        preferred_element_type=jnp.float32,
    )
    si = jax.lax.bitcast_convert_type(logits_t, jnp.int32)
    sortable = si ^ (jax.lax.shift_right_arithmetic(si, 31) & 0x7FFFFFFF)
    rev_iota = _IMASK - jax.lax.broadcasted_iota(
        jnp.int32, (NUM_EXPERTS, BLOCK), 0)
    keys_ref[...] = (sortable & ~_IMASK) | rev_iota


def _tc_keys(x, w_gate):
    return pl.pallas_call(
        _keys_body,
        grid=(TOKENS // BLOCK,),
        in_specs=[
            pl.BlockSpec((BLOCK, N_EMBD), lambda i: (i, 0)),
            pl.BlockSpec((NUM_EXPERTS, N_EMBD), lambda i: (0, 0)),
        ],
        out_specs=pl.BlockSpec((NUM_EXPERTS, BLOCK), lambda i: (0, i)),
        out_shape=jax.ShapeDtypeStruct((NUM_EXPERTS, TOKENS), jnp.int32),
    )(x, w_gate)


_SC_MESH = plsc.VectorSubcoreMesh(
    core_axis_name="c", subcore_axis_name="s", num_cores=NC, num_subcores=NS)


@functools.partial(
    pl.kernel,
    out_type=(
        jax.ShapeDtypeStruct((TOP_K, TOKENS), jnp.int32),
        jax.ShapeDtypeStruct((TOP_K, TOKENS), jnp.float32),
    ),
    mesh=_SC_MESH,
    scratch_types=[
        pltpu.VMEM((2, NUM_EXPERTS, SLAB), jnp.int32),
        pltpu.VMEM((TOP_K, TOK_PER_W), jnp.int32),
        pltpu.VMEM((TOP_K, TOK_PER_W), jnp.float32),
        pltpu.SemaphoreType.DMA,
        pltpu.SemaphoreType.DMA,
    ],
)
def _sc_topk(keys_hbm, idx_hbm, score_hbm, keys_v, idx_v, score_v,
             sem0, sem1):
    wid = lax.axis_index("s") * NC + lax.axis_index("c")
    base = wid * TOK_PER_W
    sems = (sem0, sem1)

    def start_slab(s):
        return pltpu.async_copy(
            keys_hbm.at[:, pl.ds(base + s * SLAB, SLAB)],
            keys_v.at[s % 2], sems[s % 2])

    copies = {0: start_slab(0)}
    for s in range(NSLAB):
        copies[s].wait()
        if s + 1 < NSLAB:
            copies[s + 1] = start_slab(s + 1)
        buf = s % 2

        @plsc.parallel_loop(0, NGROUP, unroll=2)
        def group(g, buf=buf, s=s):
            off = g * L
            out_off = s * SLAB + off
            best = [keys_v[buf, e, pl.ds(off, L)] for e in range(TOP_K)]
            for (a, b) in _NET8:
                hi = jnp.maximum(best[a], best[b])
                best[b] = jnp.minimum(best[a], best[b])
                best[a] = hi
            for e in range(TOP_K, NUM_EXPERTS):
                v = keys_v[buf, e, pl.ds(off, L)]
                for j in range(TOP_K):
                    hi = jnp.maximum(best[j], v)
                    v = jnp.minimum(best[j], v)
                    best[j] = hi
            vals = []
            for j in range(TOP_K):
                k = best[j]
                idx_v[j, pl.ds(out_off, L)] = _IMASK - (k & _IMASK)
                vs = k & ~_IMASK
                vsi = vs ^ (lax.shift_right_arithmetic(vs, 31) & 0x7FFFFFFF)
                vals.append(lax.bitcast_convert_type(vsi, jnp.float32))
            exps = [jnp.exp(v - vals[0]) for v in vals]
            tot = exps[0]
            for j in range(1, TOP_K):
                tot = tot + exps[j]
            for j in range(TOP_K):
                score_v[j, pl.ds(out_off, L)] = exps[j] / tot

    pltpu.sync_copy(idx_v, idx_hbm.at[:, pl.ds(base, TOK_PER_W)])
    pltpu.sync_copy(score_v, score_hbm.at[:, pl.ds(base, TOK_PER_W)])


@jax.jit
def kernel(x, w_gate):
    keys = _tc_keys(x, w_gate)
    idx_t, score_t = _sc_topk(keys)
    return idx_t.T, score_t.T


# speedup vs baseline: 1.0412x; 1.0279x over previous
import functools

import jax
import jax.numpy as jnp
from jax import lax
from jax.experimental import pallas as pl
from jax.experimental.pallas import tpu as pltpu
from jax.experimental.pallas import tpu_sc as plsc

N_EMBD = 768
NUM_EXPERTS = 64
TOP_K = 8
TOKENS = 32768
BLOCK = 4096

NC, NS, L = 2, 16, 16
NW = NC * NS
TOK_PER_W = TOKENS // NW
SLAB = 256
NSLAB = TOK_PER_W // SLAB
NGROUP = SLAB // L

_IMASK = NUM_EXPERTS - 1

_NET8 = ((0, 1), (2, 3), (4, 5), (6, 7), (0, 2), (1, 3), (4, 6), (5, 7),
         (1, 2), (5, 6), (0, 4), (3, 7), (1, 5), (2, 6), (1, 4), (3, 6),
         (2, 4), (3, 5), (3, 4))


def _keys_body(x_ref, w_ref, keys_ref):
    logits_t = jax.lax.dot_general(
        w_ref[...], x_ref[...],
        dimension_numbers=(((1,), (1,)), ((), ())),
        preferred_element_type=jnp.float32,
    )
    si = jax.lax.bitcast_convert_type(logits_t, jnp.int32)
    sortable = si ^ (jax.lax.shift_right_arithmetic(si, 31) & 0x7FFFFFFF)
    rev_iota = _IMASK - jax.lax.broadcasted_iota(
        jnp.int32, (NUM_EXPERTS, BLOCK), 0)
    keys_ref[...] = (sortable & ~_IMASK) | rev_iota


def _tc_keys(x, w_gate):
    return pl.pallas_call(
        _keys_body,
        grid=(TOKENS // BLOCK,),
        in_specs=[
            pl.BlockSpec((BLOCK, N_EMBD), lambda i: (i, 0)),
            pl.BlockSpec((NUM_EXPERTS, N_EMBD), lambda i: (0, 0)),
        ],
        out_specs=pl.BlockSpec((NUM_EXPERTS, BLOCK), lambda i: (0, i)),
        out_shape=jax.ShapeDtypeStruct((NUM_EXPERTS, TOKENS), jnp.int32),
    )(x, w_gate)


_SC_MESH = plsc.VectorSubcoreMesh(
    core_axis_name="c", subcore_axis_name="s", num_cores=NC, num_subcores=NS)


@functools.partial(
    pl.kernel,
    out_type=(
        jax.ShapeDtypeStruct((TOP_K, TOKENS), jnp.int32),
        jax.ShapeDtypeStruct((TOP_K, TOKENS), jnp.float32),
    ),
    mesh=_SC_MESH,
    scratch_types=[
        pltpu.VMEM((2, NUM_EXPERTS, SLAB), jnp.int32),
        pltpu.VMEM((TOP_K, TOK_PER_W), jnp.int32),
        pltpu.VMEM((TOP_K, TOK_PER_W), jnp.float32),
        pltpu.SemaphoreType.DMA,
        pltpu.SemaphoreType.DMA,
    ],
)
def _sc_topk(keys_hbm, idx_hbm, score_hbm, keys_v, idx_v, score_v,
             sem0, sem1):
    wid = lax.axis_index("s") * NC + lax.axis_index("c")
    base = wid * TOK_PER_W
    sems = (sem0, sem1)

    def start_slab(s):
        return pltpu.async_copy(
            keys_hbm.at[:, pl.ds(base + s * SLAB, SLAB)],
            keys_v.at[s % 2], sems[s % 2])

    copies = {0: start_slab(0)}
    for s in range(NSLAB):
        copies[s].wait()
        if s + 1 < NSLAB:
            copies[s + 1] = start_slab(s + 1)
        buf = s % 2

        def group(g, carry, buf=buf, s=s):
            off = g * L
            out_off = s * SLAB + off
            best = [keys_v[buf, e, pl.ds(off, L)] for e in range(TOP_K)]
            for (a, b) in _NET8:
                hi = jnp.maximum(best[a], best[b])
                best[b] = jnp.minimum(best[a], best[b])
                best[a] = hi
            for e in range(TOP_K, NUM_EXPERTS):
                v = keys_v[buf, e, pl.ds(off, L)]
                for j in range(TOP_K):
                    hi = jnp.maximum(best[j], v)
                    v = jnp.minimum(best[j], v)
                    best[j] = hi
            vals = []
            for j in range(TOP_K):
                k = best[j]
                idx_v[j, pl.ds(out_off, L)] = _IMASK - (k & _IMASK)
                vs = k & ~_IMASK
                vsi = vs ^ (lax.shift_right_arithmetic(vs, 31) & 0x7FFFFFFF)
                vals.append(lax.bitcast_convert_type(vsi, jnp.float32))
            exps = [jnp.exp(v - vals[0]) for v in vals]
            tot = exps[0]
            for j in range(1, TOP_K):
                tot = tot + exps[j]
            for j in range(TOP_K):
                score_v[j, pl.ds(out_off, L)] = exps[j] / tot
            return carry

        lax.fori_loop(0, NGROUP, group, 0)

    pltpu.sync_copy(idx_v, idx_hbm.at[:, pl.ds(base, TOK_PER_W)])
    pltpu.sync_copy(score_v, score_hbm.at[:, pl.ds(base, TOK_PER_W)])


@jax.jit
def kernel(x, w_gate):
    keys = _tc_keys(x, w_gate)
    idx_t, score_t = _sc_topk(keys)
    return idx_t.T, score_t.T


# speedup vs baseline: 1.1462x; 1.1009x over previous
import functools

import jax
import jax.numpy as jnp
from jax import lax
from jax.experimental import pallas as pl
from jax.experimental.pallas import tpu as pltpu
from jax.experimental.pallas import tpu_sc as plsc

N_EMBD = 768
NUM_EXPERTS = 64
TOP_K = 8
TOKENS = 32768
BLOCK = 4096

NC, NS, L = 2, 16, 16
NW = NC * NS
TOK_PER_W = TOKENS // NW
SLAB = 256
NSLAB = TOK_PER_W // SLAB
NGROUP = SLAB // L

_IMASK = NUM_EXPERTS - 1

def _ce(a, b):
    return jnp.maximum(a, b), jnp.minimum(a, b)


def _merge22(a, b):
    r0, m1 = _ce(a[0], b[0])
    m2, r3 = _ce(a[1], b[1])
    r1, r2 = _ce(m1, m2)
    return [r0, r1, r2, r3]


def _merge44(a, b):
    e = _merge22([a[0], a[2]], [b[0], b[2]])
    o = _merge22([a[1], a[3]], [b[1], b[3]])
    r = [e[0]]
    for i in range(3):
        hi, lo = _ce(o[i], e[i + 1])
        r += [hi, lo]
    r.append(o[3])
    return r


def _capmerge88(a, b):
    t = [jnp.maximum(a[i], b[7 - i]) for i in range(8)]
    for i in range(4):
        t[i], t[i + 4] = _ce(t[i], t[i + 4])
    for i in (0, 1, 4, 5):
        t[i], t[i + 2] = _ce(t[i], t[i + 2])
    for i in (0, 2, 4, 6):
        t[i], t[i + 1] = _ce(t[i], t[i + 1])
    return t


def _top8of16(v):
    s2 = [_ce(v[2 * i], v[2 * i + 1]) for i in range(8)]
    s4 = [_merge22(s2[2 * i], s2[2 * i + 1]) for i in range(4)]
    s8 = [_merge44(s4[2 * i], s4[2 * i + 1]) for i in range(2)]
    return _capmerge88(s8[0], s8[1])


def _keys_body(x_ref, w_ref, keys_ref):
    logits_t = jax.lax.dot_general(
        w_ref[...], x_ref[...],
        dimension_numbers=(((1,), (1,)), ((), ())),
        preferred_element_type=jnp.float32,
    )
    si = jax.lax.bitcast_convert_type(logits_t, jnp.int32)
    sortable = si ^ (jax.lax.shift_right_arithmetic(si, 31) & 0x7FFFFFFF)
    rev_iota = _IMASK - jax.lax.broadcasted_iota(
        jnp.int32, (NUM_EXPERTS, BLOCK), 0)
    keys_ref[...] = (sortable & ~_IMASK) | rev_iota


def _tc_keys(x, w_gate):
    return pl.pallas_call(
        _keys_body,
        grid=(TOKENS // BLOCK,),
        in_specs=[
            pl.BlockSpec((BLOCK, N_EMBD), lambda i: (i, 0)),
            pl.BlockSpec((NUM_EXPERTS, N_EMBD), lambda i: (0, 0)),
        ],
        out_specs=pl.BlockSpec((NUM_EXPERTS, BLOCK), lambda i: (0, i)),
        out_shape=jax.ShapeDtypeStruct((NUM_EXPERTS, TOKENS), jnp.int32),
    )(x, w_gate)


_SC_MESH = plsc.VectorSubcoreMesh(
    core_axis_name="c", subcore_axis_name="s", num_cores=NC, num_subcores=NS)


@functools.partial(
    pl.kernel,
    out_type=(
        jax.ShapeDtypeStruct((TOP_K, TOKENS), jnp.int32),
        jax.ShapeDtypeStruct((TOP_K, TOKENS), jnp.float32),
    ),
    mesh=_SC_MESH,
    scratch_types=[
        pltpu.VMEM((2, NUM_EXPERTS, SLAB), jnp.int32),
        pltpu.VMEM((TOP_K, TOK_PER_W), jnp.int32),
        pltpu.VMEM((TOP_K, TOK_PER_W), jnp.float32),
        pltpu.SemaphoreType.DMA,
        pltpu.SemaphoreType.DMA,
    ],
)
def _sc_topk(keys_hbm, idx_hbm, score_hbm, keys_v, idx_v, score_v,
             sem0, sem1):
    wid = lax.axis_index("s") * NC + lax.axis_index("c")
    base = wid * TOK_PER_W
    sems = (sem0, sem1)

    def start_slab(s):
        return pltpu.async_copy(
            keys_hbm.at[:, pl.ds(base + s * SLAB, SLAB)],
            keys_v.at[s % 2], sems[s % 2])

    copies = {0: start_slab(0)}
    for s in range(NSLAB):
        copies[s].wait()
        if s + 1 < NSLAB:
            copies[s + 1] = start_slab(s + 1)
        buf = s % 2

        def group(g, carry, buf=buf, s=s):
            off = g * L
            out_off = s * SLAB + off
            blocks = []
            for blk in range(NUM_EXPERTS // 16):
                v = [keys_v[buf, 16 * blk + e, pl.ds(off, L)]
                     for e in range(16)]
                blocks.append(_top8of16(v))
            best = _capmerge88(_capmerge88(blocks[0], blocks[1]),
                               _capmerge88(blocks[2], blocks[3]))
            vals = []
            for j in range(TOP_K):
                k = best[j]
                idx_v[j, pl.ds(out_off, L)] = _IMASK - (k & _IMASK)
                vs = k & ~_IMASK
                vsi = vs ^ (lax.shift_right_arithmetic(vs, 31) & 0x7FFFFFFF)
                vals.append(lax.bitcast_convert_type(vsi, jnp.float32))
            exps = [jnp.exp(v - vals[0]) for v in vals]
            tot = exps[0]
            for j in range(1, TOP_K):
                tot = tot + exps[j]
            for j in range(TOP_K):
                score_v[j, pl.ds(out_off, L)] = exps[j] / tot
            return carry

        lax.fori_loop(0, NGROUP, group, 0)

    pltpu.sync_copy(idx_v, idx_hbm.at[:, pl.ds(base, TOK_PER_W)])
    pltpu.sync_copy(score_v, score_hbm.at[:, pl.ds(base, TOK_PER_W)])


@jax.jit
def kernel(x, w_gate):
    keys = _tc_keys(x, w_gate)
    idx_t, score_t = _sc_topk(keys)
    return idx_t.T, score_t.T


# speedup vs baseline: 1.1516x; 1.0047x over previous
import functools

import jax
import jax.numpy as jnp
from jax import lax
from jax.experimental import pallas as pl
from jax.experimental.pallas import tpu as pltpu
from jax.experimental.pallas import tpu_sc as plsc

N_EMBD = 768
NUM_EXPERTS = 64
TOP_K = 8
TOKENS = 32768
BLOCK = 4096

NC, NS, L = 2, 16, 16
NW = NC * NS
TOK_PER_W = TOKENS // NW
SLAB = 512
NSLAB = TOK_PER_W // SLAB
NGROUP = SLAB // L

_IMASK = NUM_EXPERTS - 1

def _ce(a, b):
    return jnp.maximum(a, b), jnp.minimum(a, b)


def _merge22(a, b):
    r0, m1 = _ce(a[0], b[0])
    m2, r3 = _ce(a[1], b[1])
    r1, r2 = _ce(m1, m2)
    return [r0, r1, r2, r3]


def _merge44(a, b):
    e = _merge22([a[0], a[2]], [b[0], b[2]])
    o = _merge22([a[1], a[3]], [b[1], b[3]])
    r = [e[0]]
    for i in range(3):
        hi, lo = _ce(o[i], e[i + 1])
        r += [hi, lo]
    r.append(o[3])
    return r


def _capmerge88(a, b):
    t = [jnp.maximum(a[i], b[7 - i]) for i in range(8)]
    for i in range(4):
        t[i], t[i + 4] = _ce(t[i], t[i + 4])
    for i in (0, 1, 4, 5):
        t[i], t[i + 2] = _ce(t[i], t[i + 2])
    for i in (0, 2, 4, 6):
        t[i], t[i + 1] = _ce(t[i], t[i + 1])
    return t


def _top8of16(v):
    s2 = [_ce(v[2 * i], v[2 * i + 1]) for i in range(8)]
    s4 = [_merge22(s2[2 * i], s2[2 * i + 1]) for i in range(4)]
    s8 = [_merge44(s4[2 * i], s4[2 * i + 1]) for i in range(2)]
    return _capmerge88(s8[0], s8[1])


def _keys_body(x_ref, w_ref, keys_ref):
    logits_t = jax.lax.dot_general(
        w_ref[...], x_ref[...],
        dimension_numbers=(((1,), (1,)), ((), ())),
        preferred_element_type=jnp.float32,
    )
    si = jax.lax.bitcast_convert_type(logits_t, jnp.int32)
    sortable = si ^ (jax.lax.shift_right_arithmetic(si, 31) & 0x7FFFFFFF)
    rev_iota = _IMASK - jax.lax.broadcasted_iota(
        jnp.int32, (NUM_EXPERTS, BLOCK), 0)
    keys_ref[...] = (sortable & ~_IMASK) | rev_iota


def _tc_keys(x, w_gate):
    return pl.pallas_call(
        _keys_body,
        grid=(TOKENS // BLOCK,),
        in_specs=[
            pl.BlockSpec((BLOCK, N_EMBD), lambda i: (i, 0)),
            pl.BlockSpec((NUM_EXPERTS, N_EMBD), lambda i: (0, 0)),
        ],
        out_specs=pl.BlockSpec((NUM_EXPERTS, BLOCK), lambda i: (0, i)),
        out_shape=jax.ShapeDtypeStruct((NUM_EXPERTS, TOKENS), jnp.int32),
    )(x, w_gate)


_SC_MESH = plsc.VectorSubcoreMesh(
    core_axis_name="c", subcore_axis_name="s", num_cores=NC, num_subcores=NS)


@functools.partial(
    pl.kernel,
    out_type=(
        jax.ShapeDtypeStruct((TOP_K, TOKENS), jnp.int32),
        jax.ShapeDtypeStruct((TOP_K, TOKENS), jnp.float32),
    ),
    mesh=_SC_MESH,
    scratch_types=[
        pltpu.VMEM((2, NUM_EXPERTS, SLAB), jnp.int32),
        pltpu.VMEM((TOP_K, TOK_PER_W), jnp.int32),
        pltpu.VMEM((TOP_K, TOK_PER_W), jnp.float32),
        pltpu.SemaphoreType.DMA,
        pltpu.SemaphoreType.DMA,
    ],
)
def _sc_topk(keys_hbm, idx_hbm, score_hbm, keys_v, idx_v, score_v,
             sem0, sem1):
    wid = lax.axis_index("s") * NC + lax.axis_index("c")
    base = wid * TOK_PER_W
    sems = (sem0, sem1)

    def start_slab(s):
        return pltpu.async_copy(
            keys_hbm.at[:, pl.ds(base + s * SLAB, SLAB)],
            keys_v.at[s % 2], sems[s % 2])

    copies = {0: start_slab(0)}
    for s in range(NSLAB):
        copies[s].wait()
        if s + 1 < NSLAB:
            copies[s + 1] = start_slab(s + 1)
        buf = s % 2

        def group(g, carry, buf=buf, s=s):
            off = g * L
            out_off = s * SLAB + off
            blocks = []
            for blk in range(NUM_EXPERTS // 16):
                v = [keys_v[buf, 16 * blk + e, pl.ds(off, L)]
                     for e in range(16)]
                blocks.append(_top8of16(v))
            best = _capmerge88(_capmerge88(blocks[0], blocks[1]),
                               _capmerge88(blocks[2], blocks[3]))
            vals = []
            for j in range(TOP_K):
                k = best[j]
                idx_v[j, pl.ds(out_off, L)] = _IMASK - (k & _IMASK)
                vs = k & ~_IMASK
                vsi = vs ^ (lax.shift_right_arithmetic(vs, 31) & 0x7FFFFFFF)
                vals.append(lax.bitcast_convert_type(vsi, jnp.float32))
            exps = [jnp.exp(v - vals[0]) for v in vals]
            tot = exps[0]
            for j in range(1, TOP_K):
                tot = tot + exps[j]
            for j in range(TOP_K):
                score_v[j, pl.ds(out_off, L)] = exps[j] / tot
            return carry

        lax.fori_loop(0, NGROUP, group, 0)

    pltpu.sync_copy(idx_v, idx_hbm.at[:, pl.ds(base, TOK_PER_W)])
    pltpu.sync_copy(score_v, score_hbm.at[:, pl.ds(base, TOK_PER_W)])


@jax.jit
def kernel(x, w_gate):
    keys = _tc_keys(x, w_gate)
    idx_t, score_t = _sc_topk(keys)
    return idx_t.T, score_t.T


# speedup vs baseline: 1.1519x; 1.0003x over previous
import functools

import jax
import jax.numpy as jnp
from jax import lax
from jax.experimental import pallas as pl
from jax.experimental.pallas import tpu as pltpu
from jax.experimental.pallas import tpu_sc as plsc

N_EMBD = 768
NUM_EXPERTS = 64
TOP_K = 8
TOKENS = 32768
BLOCK = 4096

NC, NS, L = 2, 16, 16
NW = NC * NS
TOK_PER_W = TOKENS // NW
SLAB = 512
NSLAB = TOK_PER_W // SLAB
NGROUP = SLAB // L

_IMASK = NUM_EXPERTS - 1

def _ce(a, b):
    return jnp.maximum(a, b), jnp.minimum(a, b)


def _merge22(a, b):
    r0, m1 = _ce(a[0], b[0])
    m2, r3 = _ce(a[1], b[1])
    r1, r2 = _ce(m1, m2)
    return [r0, r1, r2, r3]


def _merge44(a, b):
    e = _merge22([a[0], a[2]], [b[0], b[2]])
    o = _merge22([a[1], a[3]], [b[1], b[3]])
    r = [e[0]]
    for i in range(3):
        hi, lo = _ce(o[i], e[i + 1])
        r += [hi, lo]
    r.append(o[3])
    return r


def _capmerge88(a, b):
    t = [jnp.maximum(a[i], b[7 - i]) for i in range(8)]
    for i in range(4):
        t[i], t[i + 4] = _ce(t[i], t[i + 4])
    for i in (0, 1, 4, 5):
        t[i], t[i + 2] = _ce(t[i], t[i + 2])
    for i in (0, 2, 4, 6):
        t[i], t[i + 1] = _ce(t[i], t[i + 1])
    return t


def _top8of16(v):
    s2 = [_ce(v[2 * i], v[2 * i + 1]) for i in range(8)]
    s4 = [_merge22(s2[2 * i], s2[2 * i + 1]) for i in range(4)]
    s8 = [_merge44(s4[2 * i], s4[2 * i + 1]) for i in range(2)]
    return _capmerge88(s8[0], s8[1])


def _keys_body(x_ref, w_ref, keys_ref):
    logits_t = jax.lax.dot_general(
        w_ref[...], x_ref[...],
        dimension_numbers=(((1,), (1,)), ((), ())),
        preferred_element_type=jnp.float32,
    )
    si = jax.lax.bitcast_convert_type(logits_t, jnp.int32)
    sortable = si ^ (jax.lax.shift_right_arithmetic(si, 31) & 0x7FFFFFFF)
    rev_iota = _IMASK - jax.lax.broadcasted_iota(
        jnp.int32, (NUM_EXPERTS, BLOCK), 0)
    keys_ref[...] = (sortable & ~_IMASK) | rev_iota


def _tc_keys(x, w_gate):
    return pl.pallas_call(
        _keys_body,
        grid=(TOKENS // BLOCK,),
        in_specs=[
            pl.BlockSpec((BLOCK, N_EMBD), lambda i: (i, 0)),
            pl.BlockSpec((NUM_EXPERTS, N_EMBD), lambda i: (0, 0)),
        ],
        out_specs=pl.BlockSpec((NUM_EXPERTS, BLOCK), lambda i: (0, i)),
        out_shape=jax.ShapeDtypeStruct((NUM_EXPERTS, TOKENS), jnp.int32),
    )(x, w_gate)


@functools.lru_cache(maxsize=None)
def _build_sc_topk():
    mesh = plsc.VectorSubcoreMesh(
        core_axis_name="c", subcore_axis_name="s",
        num_cores=NC, num_subcores=NS)
    return functools.partial(
        pl.kernel,
        out_type=(
            jax.ShapeDtypeStruct((TOP_K, TOKENS), jnp.int32),
            jax.ShapeDtypeStruct((TOP_K, TOKENS), jnp.float32),
        ),
        mesh=mesh,
        scratch_types=[
            pltpu.VMEM((2, NUM_EXPERTS, SLAB), jnp.int32),
            pltpu.VMEM((TOP_K, TOK_PER_W), jnp.int32),
            pltpu.VMEM((TOP_K, TOK_PER_W), jnp.float32),
            pltpu.SemaphoreType.DMA,
            pltpu.SemaphoreType.DMA,
        ],
    )(_sc_topk)


def _sc_topk(keys_hbm, idx_hbm, score_hbm, keys_v, idx_v, score_v,
             sem0, sem1):
    wid = lax.axis_index("s") * NC + lax.axis_index("c")
    base = wid * TOK_PER_W
    sems = (sem0, sem1)

    def start_slab(s):
        return pltpu.async_copy(
            keys_hbm.at[:, pl.ds(base + s * SLAB, SLAB)],
            keys_v.at[s % 2], sems[s % 2])

    copies = {0: start_slab(0)}
    for s in range(NSLAB):
        copies[s].wait()
        if s + 1 < NSLAB:
            copies[s + 1] = start_slab(s + 1)
        buf = s % 2

        def group(g, carry, buf=buf, s=s):
            off = g * L
            out_off = s * SLAB + off
            blocks = []
            for blk in range(NUM_EXPERTS // 16):
                v = [keys_v[buf, 16 * blk + e, pl.ds(off, L)]
                     for e in range(16)]
                blocks.append(_top8of16(v))
            best = _capmerge88(_capmerge88(blocks[0], blocks[1]),
                               _capmerge88(blocks[2], blocks[3]))
            vals = []
            for j in range(TOP_K):
                k = best[j]
                idx_v[j, pl.ds(out_off, L)] = _IMASK - (k & _IMASK)
                vs = k & ~_IMASK
                vsi = vs ^ (lax.shift_right_arithmetic(vs, 31) & 0x7FFFFFFF)
                vals.append(lax.bitcast_convert_type(vsi, jnp.float32))
            exps = [jnp.exp(v - vals[0]) for v in vals]
            tot = exps[0]
            for j in range(1, TOP_K):
                tot = tot + exps[j]
            for j in range(TOP_K):
                score_v[j, pl.ds(out_off, L)] = exps[j] / tot
            return carry

        lax.fori_loop(0, NGROUP, group, 0)

    pltpu.sync_copy(idx_v, idx_hbm.at[:, pl.ds(base, TOK_PER_W)])
    pltpu.sync_copy(score_v, score_hbm.at[:, pl.ds(base, TOK_PER_W)])


@jax.jit
def kernel(x, w_gate):
    keys = _tc_keys(x, w_gate)
    idx_t, score_t = _build_sc_topk()(keys)
    return idx_t.T, score_t.T
